# baseline (speedup 1.0000x reference)
import jax
import jax.numpy as jnp
from jax.experimental import pallas as pl
from jax.experimental.pallas import tpu as pltpu

_NC = 21
_IGNORE = 20


def _half_loss(x_ref, t_ref, e_ref, cmod_ref, e2_ref):
    x = x_ref[...]
    tf = t_ref[...].astype(jnp.float32)
    t_exp = jnp.dot(tf, e_ref[...], preferred_element_type=jnp.float32)
    cmod = cmod_ref[...]
    gsel = jnp.where(cmod == t_exp, x, 0.0)
    sp = jnp.maximum(x, 0.0) + jnp.log(1.0 + jnp.exp(-jnp.abs(x)))
    contrib = sp - gsel
    p = jnp.dot(contrib, e2_ref[...], preferred_element_type=jnp.float32)
    pv = jnp.where(t_ref[...] != _IGNORE, p, 0.0)
    return jnp.sum(pv, keepdims=True)


def _bce_block_kernel(x1_ref, x2_ref, t1_ref, t2_ref, e_ref, cmod_ref,
                      e2_ref, out_ref):
    s = (_half_loss(x1_ref, t1_ref, e_ref, cmod_ref, e2_ref)
         + _half_loss(x2_ref, t2_ref, e_ref, cmod_ref, e2_ref))

    @pl.when(pl.program_id(0) == 0)
    def _init():
        out_ref[...] = jnp.zeros_like(out_ref)

    out_ref[...] += s


def kernel(inputs, targets):
    b, l = targets.shape
    n = inputs.shape[1]
    tgt = targets.astype(jnp.int32)
    grp = jnp.arange(n, dtype=jnp.int32) // _NC
    cls = jnp.arange(n, dtype=jnp.int32) % _NC
    e = (grp[None, :] == jnp.arange(l, dtype=jnp.int32)[:, None]
         ).astype(jnp.float32)
    cmod = cls[None, :].astype(jnp.float32)
    e2 = ((grp[:, None] == jnp.arange(l, dtype=jnp.int32)[None, :])
          & (cls[:, None] != _IGNORE)).astype(jnp.float32)
    tb = 1024
    half = b // tb // 2
    out = pl.pallas_call(
        _bce_block_kernel,
        grid=(half,),
        in_specs=[
            pl.BlockSpec((tb, n), lambda i: (i, 0)),
            pl.BlockSpec((tb, n), lambda i, _h=half: (i + _h, 0)),
            pl.BlockSpec((tb, l), lambda i: (i, 0)),
            pl.BlockSpec((tb, l), lambda i, _h=half: (i + _h, 0)),
            pl.BlockSpec((l, n), lambda i: (0, 0)),
            pl.BlockSpec((1, n), lambda i: (0, 0)),
            pl.BlockSpec((n, l), lambda i: (0, 0)),
        ],
        out_specs=pl.BlockSpec((1, 1), lambda i: (0, 0)),
        out_shape=jax.ShapeDtypeStruct((1, 1), jnp.float32),
        compiler_params=pltpu.CompilerParams(
            dimension_semantics=("arbitrary",)),
    )(inputs, inputs, tgt, tgt, e, cmod, e2)
    return out[0, 0]


# speedup vs baseline: 1.0109x; 1.0109x over previous
import jax
import jax.numpy as jnp
from jax.experimental import pallas as pl
from jax.experimental.pallas import tpu as pltpu

_NC = 21
_IGNORE = 20


def _half_loss(x_ref, t_ref, e_ref, cmod_ref, e2_ref):
    x = x_ref[...]
    tf = t_ref[...].astype(jnp.float32)
    t_exp = jnp.dot(tf, e_ref[...], preferred_element_type=jnp.float32)
    cmod = cmod_ref[...]
    gsel = jnp.where(cmod == t_exp, x, 0.0)
    sp = jnp.maximum(x, 0.0) + jnp.log(1.0 + jnp.exp(-jnp.abs(x)))
    contrib = sp - gsel
    p = jnp.dot(contrib, e2_ref[...], preferred_element_type=jnp.float32)
    pv = jnp.where(t_ref[...] != _IGNORE, p, 0.0)
    return jnp.sum(pv, keepdims=True)


def _bce_block_kernel(x1_ref, x2_ref, t1_ref, t2_ref, e_ref, cmod_ref,
                      e2_ref, out_ref):
    s = (_half_loss(x1_ref, t1_ref, e_ref, cmod_ref, e2_ref)
         + _half_loss(x2_ref, t2_ref, e_ref, cmod_ref, e2_ref))

    @pl.when(pl.program_id(0) == 0)
    def _init():
        out_ref[...] = jnp.zeros_like(out_ref)

    out_ref[...] += s


def kernel(inputs, targets):
    b, l = targets.shape
    n = inputs.shape[1]
    tgt = targets.astype(jnp.int32)
    grp = jnp.arange(n, dtype=jnp.int32) // _NC
    cls = jnp.arange(n, dtype=jnp.int32) % _NC
    e = (grp[None, :] == jnp.arange(l, dtype=jnp.int32)[:, None]
         ).astype(jnp.float32)
    cmod = cls[None, :].astype(jnp.float32)
    e2 = ((grp[:, None] == jnp.arange(l, dtype=jnp.int32)[None, :])
          & (cls[:, None] != _IGNORE)).astype(jnp.float32)
    tb = 512
    half = b // tb // 2
    out = pl.pallas_call(
        _bce_block_kernel,
        grid=(half,),
        in_specs=[
            pl.BlockSpec((tb, n), lambda i: (i, 0)),
            pl.BlockSpec((tb, n), lambda i, _h=half: (i + _h, 0)),
            pl.BlockSpec((tb, l), lambda i: (i, 0)),
            pl.BlockSpec((tb, l), lambda i, _h=half: (i + _h, 0)),
            pl.BlockSpec((l, n), lambda i: (0, 0)),
            pl.BlockSpec((1, n), lambda i: (0, 0)),
            pl.BlockSpec((n, l), lambda i: (0, 0)),
        ],
        out_specs=pl.BlockSpec((1, 1), lambda i: (0, 0)),
        out_shape=jax.ShapeDtypeStruct((1, 1), jnp.float32),
        compiler_params=pltpu.CompilerParams(
            dimension_semantics=("arbitrary",)),
    )(inputs, inputs, tgt, tgt, e, cmod, e2)
    return out[0, 0]
